# baseline (speedup 1.0000x reference)
import functools
import jax
import jax.numpy as jnp
from jax import lax
from jax.experimental import pallas as pl
from jax.experimental.pallas import tpu as pltpu
from jax.experimental.pallas import tpu_sc as plsc

_B = 16384
_D = 64
_NC = 2
_NS = 16
_NW = _NC * _NS
_BPW = _B // _NW
_LANES = 16
_VPR = _D // _LANES
_K = 16
_CH = 256
_NCH = _BPW // _CH


def _sc_body(tensor_hbm, targets_hbm, centers_hbm, out_hbm,
             idx_v, rows_v, t_v, acc_v, sems):
  wid = lax.axis_index("s") * _NC + lax.axis_index("c")
  base = wid * _BPW

  pltpu.sync_copy(targets_hbm.at[pl.ds(base, _BPW)], idx_v)

  acc = jnp.zeros((_LANES,), jnp.float32)
  for c in range(_NCH):
    pltpu.sync_copy(tensor_hbm.at[pl.ds(base + c * _CH, _CH), :], t_v)

    def burst(o, _):
      vec = idx_v[pl.ds(c * _CH + o * _K, _K)]
      descs = []
      for k in range(_K):
        r = o * _K + k
        idx = vec[k]
        d = pltpu.async_copy(centers_hbm.at[idx], rows_v.at[r], sems.at[k])
        descs.append(d)
      for d in descs:
        d.wait()
      return 0

    lax.fori_loop(0, _CH // _K, burst, 0)

    def row_step(r, a):
      for j in range(_VPR):
        d = (t_v[r, pl.ds(j * _LANES, _LANES)]
             - rows_v[r, pl.ds(j * _LANES, _LANES)])
        a = a + d * d
      return a

    acc = lax.fori_loop(0, _CH, row_step, acc)

  acc_v[...] = acc
  pltpu.sync_copy(acc_v, out_hbm.at[wid])


@jax.jit
def kernel(tensor, targets, centers):
  targets = targets.astype(jnp.int32)
  partials = pl.kernel(
      _sc_body,
      out_type=jax.ShapeDtypeStruct((_NW, _LANES), jnp.float32),
      mesh=plsc.VectorSubcoreMesh(core_axis_name="c", subcore_axis_name="s"),
      scratch_types=[
          pltpu.VMEM((_BPW,), jnp.int32),
          pltpu.VMEM((_CH, _D), jnp.float32),
          pltpu.VMEM((_CH, _D), jnp.float32),
          pltpu.VMEM((_LANES,), jnp.float32),
          pltpu.SemaphoreType.DMA((_K,)),
      ],
  )(tensor, targets, centers)
  return 0.5 * jnp.sum(partials)


# speedup vs baseline: 1.4333x; 1.4333x over previous
import jax
import jax.numpy as jnp
from jax import lax
from jax.experimental import pallas as pl
from jax.experimental.pallas import tpu as pltpu
from jax.experimental.pallas import tpu_sc as plsc

_B = 16384
_D = 64
_NC = 2
_NS = 16
_NW = _NC * _NS
_BPW = _B // _NW
_LANES = 16
_VPR = _D // _LANES
_CH = 128
_NCH = _BPW // _CH
_NBUF = 3


def _sc_body(tensor_hbm, targets_hbm, centers_hbm, out_hbm,
             idx_v, rows_v, t_v, acc_v, gsems, tsem):
  wid = lax.axis_index("s") * _NC + lax.axis_index("c")
  base = wid * _BPW

  pltpu.sync_copy(targets_hbm.at[pl.ds(base, _BPW)], idx_v)
  t_copy = pltpu.async_copy(tensor_hbm.at[pl.ds(base, _BPW), :], t_v, tsem)

  def issue_chunk(c):
    descs = []
    buf = c % _NBUF
    for o in range(_CH // _LANES):
      vec = idx_v[pl.ds(c * _CH + o * _LANES, _LANES)]
      for k in range(_LANES):
        r = o * _LANES + k
        descs.append(pltpu.async_copy(
            centers_hbm.at[vec[k]], rows_v.at[buf * _CH + r], gsems.at[buf]))
    return descs

  descs = [None] * _NCH
  descs[0] = issue_chunk(0)
  descs[1] = issue_chunk(1)
  t_copy.wait()

  acc = jnp.zeros((_LANES,), jnp.float32)
  for c in range(_NCH):
    for d in descs[c]:
      d.wait()
    if c + 2 < _NCH:
      descs[c + 2] = issue_chunk(c + 2)

    buf = c % _NBUF

    def row_step(r, a):
      for j in range(_VPR):
        d = (t_v[c * _CH + r, pl.ds(j * _LANES, _LANES)]
             - rows_v[buf * _CH + r, pl.ds(j * _LANES, _LANES)])
        a = a + d * d
      return a

    acc = lax.fori_loop(0, _CH, row_step, acc)

  acc_v[...] = acc
  pltpu.sync_copy(acc_v, out_hbm.at[wid])


@jax.jit
def kernel(tensor, targets, centers):
  targets = targets.astype(jnp.int32)
  partials = pl.kernel(
      _sc_body,
      out_type=jax.ShapeDtypeStruct((_NW, _LANES), jnp.float32),
      mesh=plsc.VectorSubcoreMesh(core_axis_name="c", subcore_axis_name="s"),
      scratch_types=[
          pltpu.VMEM((_BPW,), jnp.int32),
          pltpu.VMEM((_NBUF * _CH, _D), jnp.float32),
          pltpu.VMEM((_BPW, _D), jnp.float32),
          pltpu.VMEM((_LANES,), jnp.float32),
          pltpu.SemaphoreType.DMA((_NBUF,)),
          pltpu.SemaphoreType.DMA,
      ],
  )(tensor, targets, centers)
  return 0.5 * jnp.sum(partials)


# speedup vs baseline: 2.1934x; 1.5303x over previous
import jax
import jax.numpy as jnp
from jax import lax
from jax.experimental import pallas as pl
from jax.experimental.pallas import tpu as pltpu
from jax.experimental.pallas import tpu_sc as plsc

_B = 16384
_D = 64
_N = 100000
_NC = 2
_NS = 16
_NW = _NC * _NS
_LANES = 16
_HALF = _B // 2


def _sc_body(tensor_t_hbm, targets_hbm, centers_t_hbm, out_hbm,
             plane_v, trow_v, idx_v, acc_v, psem, tsem, isem):
  wid = lax.axis_index("s") * _NC + lax.axis_index("c")

  acc = jnp.zeros((_LANES,), jnp.float32)
  for p in range(2):
    j = wid + _NW * p
    d_plane = pltpu.async_copy(centers_t_hbm.at[j], plane_v, psem)
    d_trow = pltpu.async_copy(tensor_t_hbm.at[j], trow_v, tsem)
    d_idx = pltpu.async_copy(targets_hbm.at[pl.ds(0, _HALF)], idx_v, isem)
    d_plane.wait()
    d_trow.wait()

    for h in range(2):
      d_idx.wait()

      def step(o, a):
        g16 = idx_v[pl.ds(o * _LANES, _LANES)]
        c16 = plsc.load_gather(plane_v, [g16])
        t16 = trow_v[pl.ds(h * _HALF + o * _LANES, _LANES)]
        d = t16 - c16
        return a + d * d

      acc = lax.fori_loop(0, _HALF // _LANES, step, acc)
      if h == 0:
        d_idx = pltpu.async_copy(
            targets_hbm.at[pl.ds(_HALF, _HALF)], idx_v, isem)

  acc_v[...] = acc
  pltpu.sync_copy(acc_v, out_hbm.at[wid])


@jax.jit
def kernel(tensor, targets, centers):
  targets = targets.astype(jnp.int32)
  partials = pl.kernel(
      _sc_body,
      out_type=jax.ShapeDtypeStruct((_NW, _LANES), jnp.float32),
      mesh=plsc.VectorSubcoreMesh(core_axis_name="c", subcore_axis_name="s"),
      scratch_types=[
          pltpu.VMEM((_N,), jnp.float32),
          pltpu.VMEM((_B,), jnp.float32),
          pltpu.VMEM((_HALF,), jnp.int32),
          pltpu.VMEM((_LANES,), jnp.float32),
          pltpu.SemaphoreType.DMA,
          pltpu.SemaphoreType.DMA,
          pltpu.SemaphoreType.DMA,
      ],
      compiler_params=pltpu.CompilerParams(needs_layout_passes=False),
  )(tensor.T, targets, centers.T)
  return 0.5 * jnp.sum(partials)
